# Initial kernel scaffold; baseline (speedup 1.0000x reference)
import jax
import jax.numpy as jnp
from jax.experimental import pallas as pl


def kernel(x, edge_index, W1, b1, W2, b2, Wc, bc):
    raise NotImplementedError("write your pallas kernel here")


# speedup vs baseline: 6.5869x; 6.5869x over previous
import functools

import jax
import jax.numpy as jnp
from jax import lax
from jax.experimental import pallas as pl
from jax.experimental.pallas import tpu as pltpu
from jax.experimental.pallas import tpu_sc as plsc

N_NODES = 10000
IN_DIM = 256
HID = 512
N_CLS = 64
N_EDGES = 160000

NSUB = 16
NCORE = 2
WIN = 128
WPS = 80
SPAN = 40
EPAD = NCORE * 0 + NSUB * WPS * WIN
NPAD = 10112
ROWS_PER_SUB = NPAD // NSUB
DEG_W = 128
FCHUNK = 128

_RB = 1264


def _sc_mesh():
    return plsc.VectorSubcoreMesh(core_axis_name="c", subcore_axis_name="s")


def _fill_rows(buf, value):
    rows, cols = buf.shape
    v = jnp.full((16,), value, jnp.float32)

    @pl.loop(0, rows)
    def _(i):
        for j in range(0, cols, 16):
            buf[i, pl.ds(j, 16)] = v


def _zero_acc_rows(acc_sh, zbuf, s):
    base = s * ROWS_PER_SUB
    for j in range(ROWS_PER_SUB // WIN):
        pltpu.sync_copy(zbuf, acc_sh.at[pl.ds(base + j * WIN, WIN)])
    tail = ROWS_PER_SUB % WIN
    if tail:
        pltpu.sync_copy(
            zbuf.at[pl.ds(0, tail)],
            acc_sh.at[pl.ds(base + (ROWS_PER_SUB // WIN) * WIN, tail)],
        )


@functools.partial(
    pl.kernel,
    out_type=[
        jax.ShapeDtypeStruct((NPAD, DEG_W), jnp.float32),
        jax.ShapeDtypeStruct((NPAD, DEG_W), jnp.float32),
    ],
    mesh=_sc_mesh(),
    scratch_types=[
        pltpu.VMEM((WPS, WIN), jnp.int32),
        pltpu.VMEM((WIN, DEG_W), jnp.float32),
        pltpu.VMEM_SHARED((NPAD, DEG_W), jnp.float32),
    ],
)
def _deg_kernel(src_hbm, dst_hbm, dego_hbm, degi_hbm, idx_v, fill_v, acc_sh):
    c = lax.axis_index("c")
    s = lax.axis_index("s")

    def side(idx_hbm, out_hbm):
        pltpu.sync_copy(idx_hbm.at[pl.ds(s * WPS, WPS)], idx_v)
        _fill_rows(fill_v, 0.0)
        _zero_acc_rows(acc_sh, fill_v, s)
        _fill_rows(fill_v, 1.0)
        plsc.subcore_barrier()

        @pl.loop(0, WPS)
        def _(w):
            pltpu.sync_copy(fill_v, acc_sh.at[idx_v.at[w]], add=True)

        plsc.subcore_barrier()
        base = s * ROWS_PER_SUB
        pltpu.sync_copy(
            acc_sh.at[pl.ds(base, ROWS_PER_SUB)],
            out_hbm.at[pl.ds(base, ROWS_PER_SUB)],
        )

    @pl.when(c == 0)
    def _():
        side(src_hbm, dego_hbm)

    @pl.when(c == 1)
    def _():
        side(dst_hbm, degi_hbm)


def _make_agg(num_tables):
    T = num_tables

    @functools.partial(
        pl.kernel,
        out_type=[jax.ShapeDtypeStruct((NPAD, FCHUNK), jnp.float32) for _ in range(T)],
        mesh=_sc_mesh(),
        scratch_types=[
            pltpu.VMEM((SPAN, WIN), jnp.int32),
            pltpu.VMEM((SPAN, WIN), jnp.int32),
            pltpu.VMEM((WIN, FCHUNK), jnp.float32),
            pltpu.VMEM((WIN, FCHUNK), jnp.float32),
            pltpu.VMEM_SHARED((NPAD, FCHUNK), jnp.float32),
            pltpu.SemaphoreType.DMA,
        ],
    )
    def agg(src_hbm, dst_hbm, *rest):
        tables = rest[:T]
        outrefs = rest[T:2 * T]
        src_v, dst_v, g0, g1, acc_sh, sem = rest[2 * T:]
        c = lax.axis_index("c")
        s = lax.axis_index("s")

        def run_chunk(table, out):
            _fill_rows(g0, 0.0)
            _zero_acc_rows(acc_sh, g0, s)
            plsc.subcore_barrier()

            for half in range(WPS // SPAN):
                row0 = s * WPS + half * SPAN
                pltpu.sync_copy(src_hbm.at[pl.ds(row0, SPAN)], src_v)
                pltpu.sync_copy(dst_hbm.at[pl.ds(row0, SPAN)], dst_v)
                pltpu.async_copy(table.at[src_v.at[0]], g0, sem)

                @pl.loop(0, SPAN, step=2)
                def _(w):
                    pltpu.make_async_copy(table.at[src_v.at[0]], g0, sem).wait()
                    pltpu.async_copy(table.at[src_v.at[w + 1]], g1, sem)
                    pltpu.sync_copy(g0, acc_sh.at[dst_v.at[w]], add=True)
                    pltpu.make_async_copy(table.at[src_v.at[0]], g1, sem).wait()

                    @pl.when(w + 2 < SPAN)
                    def _():
                        pltpu.async_copy(table.at[src_v.at[w + 2]], g0, sem)

                    pltpu.sync_copy(g1, acc_sh.at[dst_v.at[w + 1]], add=True)

            plsc.subcore_barrier()
            base = s * ROWS_PER_SUB
            pltpu.sync_copy(
                acc_sh.at[pl.ds(base, ROWS_PER_SUB)],
                out.at[pl.ds(base, ROWS_PER_SUB)],
            )

        for k in range(NCORE):
            @pl.when(c == k)
            def _(k=k):
                for t in range(T // 2):
                    idx = k * (T // 2) + t
                    run_chunk(tables[idx], outrefs[idx])

    return agg


_agg2 = _make_agg(2)
_agg4 = _make_agg(4)


def _row_coeff(deg_ref, i):
    rows = i * _RB + lax.broadcasted_iota(jnp.int32, (_RB, 1), 0)
    cexp = lax.rsqrt(jnp.maximum(deg_ref[:, 0:1], 1.0))
    return jnp.where(rows < N_NODES, cexp, 0.0), rows


def _scale_split_body(x_ref, dego_ref, o0_ref, o1_ref):
    i = pl.program_id(0)
    c, _ = _row_coeff(dego_ref, i)
    xs = x_ref[...] * c
    o0_ref[...] = xs[:, :FCHUNK]
    o1_ref[...] = xs[:, FCHUNK:]


_scale_split = pl.pallas_call(
    _scale_split_body,
    grid=(NPAD // _RB,),
    in_specs=[
        pl.BlockSpec((_RB, IN_DIM), lambda i: (i, 0)),
        pl.BlockSpec((_RB, DEG_W), lambda i: (i, 0)),
    ],
    out_specs=[pl.BlockSpec((_RB, FCHUNK), lambda i: (i, 0))] * 2,
    out_shape=[jax.ShapeDtypeStruct((NPAD, FCHUNK), jnp.float32)] * 2,
)


def _l1_body(a0, a1, degi, dego, w1, b1, o0, o1, o2, o3):
    i = pl.program_id(0)
    cd, _ = _row_coeff(degi, i)
    cs, _ = _row_coeff(dego, i)
    a = jnp.concatenate([a0[...], a1[...]], axis=1) * cd
    h = jnp.dot(a, w1[...], preferred_element_type=jnp.float32) + b1[...]
    hs = jnp.maximum(h, 0.0) * cs
    for t, o in enumerate((o0, o1, o2, o3)):
        o[...] = hs[:, t * FCHUNK:(t + 1) * FCHUNK]


_l1 = pl.pallas_call(
    _l1_body,
    grid=(NPAD // _RB,),
    in_specs=[
        pl.BlockSpec((_RB, FCHUNK), lambda i: (i, 0)),
        pl.BlockSpec((_RB, FCHUNK), lambda i: (i, 0)),
        pl.BlockSpec((_RB, DEG_W), lambda i: (i, 0)),
        pl.BlockSpec((_RB, DEG_W), lambda i: (i, 0)),
        pl.BlockSpec((IN_DIM, HID), lambda i: (0, 0)),
        pl.BlockSpec((1, HID), lambda i: (0, 0)),
    ],
    out_specs=[pl.BlockSpec((_RB, FCHUNK), lambda i: (i, 0))] * 4,
    out_shape=[jax.ShapeDtypeStruct((NPAD, FCHUNK), jnp.float32)] * 4,
)


def _l2_body(a0, a1, a2, a3, degi, w2, b2, wc, bc, out_ref, acc_ref):
    i = pl.program_id(0)

    @pl.when(i == 0)
    def _():
        acc_ref[...] = jnp.zeros_like(acc_ref)

    cd, rows = _row_coeff(degi, i)
    a = jnp.concatenate([a0[...], a1[...], a2[...], a3[...]], axis=1) * cd
    h = jnp.dot(a, w2[...], preferred_element_type=jnp.float32) + b2[...]
    h = jnp.maximum(h, 0.0) * jnp.where(rows < N_NODES, 1.0, 0.0)
    acc_ref[...] += jnp.sum(h, axis=0, keepdims=True)

    @pl.when(i == pl.num_programs(0) - 1)
    def _():
        hg = acc_ref[...] * (1.0 / N_NODES)
        out_ref[...] = (
            jnp.dot(hg, wc[...], preferred_element_type=jnp.float32) + bc[...]
        )


_l2 = pl.pallas_call(
    _l2_body,
    grid=(NPAD // _RB,),
    in_specs=[
        pl.BlockSpec((_RB, FCHUNK), lambda i: (i, 0)),
        pl.BlockSpec((_RB, FCHUNK), lambda i: (i, 0)),
        pl.BlockSpec((_RB, FCHUNK), lambda i: (i, 0)),
        pl.BlockSpec((_RB, FCHUNK), lambda i: (i, 0)),
        pl.BlockSpec((_RB, DEG_W), lambda i: (i, 0)),
        pl.BlockSpec((HID, HID), lambda i: (0, 0)),
        pl.BlockSpec((1, HID), lambda i: (0, 0)),
        pl.BlockSpec((HID, N_CLS), lambda i: (0, 0)),
        pl.BlockSpec((1, N_CLS), lambda i: (0, 0)),
    ],
    out_specs=pl.BlockSpec((1, N_CLS), lambda i: (0, 0)),
    out_shape=jax.ShapeDtypeStruct((1, N_CLS), jnp.float32),
    scratch_shapes=[pltpu.VMEM((1, HID), jnp.float32)],
)


def kernel(x, edge_index, W1, b1, W2, b2, Wc, bc):
    src = edge_index[0].astype(jnp.int32)
    dst = edge_index[1].astype(jnp.int32)
    pad_n = EPAD - N_EDGES
    pad_idx = N_NODES + (jnp.arange(pad_n, dtype=jnp.int32) % (NPAD - N_NODES))
    src_w = jnp.concatenate([src, pad_idx]).reshape(NSUB * WPS, WIN)
    dst_w = jnp.concatenate([dst, pad_idx]).reshape(NSUB * WPS, WIN)
    x_pad = jnp.pad(x, ((0, NPAD - N_NODES), (0, 0)))

    dego, degi = _deg_kernel(src_w, dst_w)
    xs0, xs1 = _scale_split(x_pad, dego)
    a10, a11 = _agg2(src_w, dst_w, xs0, xs1)
    h0, h1, h2, h3 = _l1(a10, a11, degi, dego, W1, b1.reshape(1, HID))
    a20, a21, a22, a23 = _agg4(src_w, dst_w, h0, h1, h2, h3)
    return _l2(a20, a21, a22, a23, degi, W2, b2.reshape(1, HID),
               Wc, bc.reshape(1, N_CLS))


# speedup vs baseline: 7.2252x; 1.0969x over previous
import functools

import jax
import jax.numpy as jnp
from jax import lax
from jax.experimental import pallas as pl
from jax.experimental.pallas import tpu as pltpu
from jax.experimental.pallas import tpu_sc as plsc

N_NODES = 10000
IN_DIM = 256
HID = 512
N_CLS = 64
N_EDGES = 160000

NSUB = 16
NCORE = 2
WIN = 128
WPS = 80
AWIN = 64
AWPS = 160
ASPAN = 40
EPAD = NCORE * 0 + NSUB * WPS * WIN
NPAD = 10112
ROWS_PER_SUB = NPAD // NSUB
DEG_W = 128
FCHUNK = 128

_RB = 1264


def _sc_mesh():
    return plsc.VectorSubcoreMesh(core_axis_name="c", subcore_axis_name="s")


def _fill_rows(buf, value):
    rows, cols = buf.shape
    v = jnp.full((16,), value, jnp.float32)

    @pl.loop(0, rows)
    def _(i):
        for j in range(0, cols, 16):
            buf[i, pl.ds(j, 16)] = v


def _zero_acc_rows(acc_sh, zbuf, s):
    zr = zbuf.shape[0]
    base = s * ROWS_PER_SUB
    for j in range(ROWS_PER_SUB // zr):
        pltpu.sync_copy(zbuf, acc_sh.at[pl.ds(base + j * zr, zr)])
    tail = ROWS_PER_SUB % zr
    if tail:
        pltpu.sync_copy(
            zbuf.at[pl.ds(0, tail)],
            acc_sh.at[pl.ds(base + (ROWS_PER_SUB // zr) * zr, tail)],
        )


@functools.partial(
    pl.kernel,
    out_type=[
        jax.ShapeDtypeStruct((NPAD, DEG_W), jnp.float32),
        jax.ShapeDtypeStruct((NPAD, DEG_W), jnp.float32),
    ],
    mesh=_sc_mesh(),
    scratch_types=[
        pltpu.VMEM((WPS, WIN), jnp.int32),
        pltpu.VMEM((WIN, DEG_W), jnp.float32),
        pltpu.VMEM_SHARED((NPAD, DEG_W), jnp.float32),
    ],
)
def _deg_kernel(src_hbm, dst_hbm, dego_hbm, degi_hbm, idx_v, fill_v, acc_sh):
    c = lax.axis_index("c")
    s = lax.axis_index("s")

    def side(idx_hbm, out_hbm):
        pltpu.sync_copy(idx_hbm.at[pl.ds(s * WPS, WPS)], idx_v)
        _fill_rows(fill_v, 0.0)
        _zero_acc_rows(acc_sh, fill_v, s)
        _fill_rows(fill_v, 1.0)
        plsc.subcore_barrier()

        @pl.loop(0, WPS)
        def _(w):
            pltpu.sync_copy(fill_v, acc_sh.at[idx_v.at[w]], add=True)

        plsc.subcore_barrier()
        base = s * ROWS_PER_SUB
        pltpu.sync_copy(
            acc_sh.at[pl.ds(base, ROWS_PER_SUB)],
            out_hbm.at[pl.ds(base, ROWS_PER_SUB)],
        )

    @pl.when(c == 0)
    def _():
        side(src_hbm, dego_hbm)

    @pl.when(c == 1)
    def _():
        side(dst_hbm, degi_hbm)


def _make_agg(num_tables):
    T = num_tables

    @functools.partial(
        pl.kernel,
        out_type=[jax.ShapeDtypeStruct((NPAD, FCHUNK), jnp.float32) for _ in range(T)],
        mesh=_sc_mesh(),
        scratch_types=[
            pltpu.VMEM((ASPAN, AWIN), jnp.int32),
            pltpu.VMEM((ASPAN, AWIN), jnp.int32),
            pltpu.VMEM((AWIN, FCHUNK), jnp.float32),
            pltpu.VMEM((AWIN, FCHUNK), jnp.float32),
            pltpu.VMEM((AWIN, FCHUNK), jnp.float32),
            pltpu.VMEM((AWIN, FCHUNK), jnp.float32),
            pltpu.VMEM_SHARED((NPAD, FCHUNK), jnp.float32),
            pltpu.SemaphoreType.DMA,
            pltpu.SemaphoreType.DMA,
        ],
    )
    def agg(src_hbm, dst_hbm, *rest):
        tables = rest[:T]
        outrefs = rest[T:2 * T]
        src_v, dst_v, g0, g1, g2, g3, acc_sh, semg, sems = rest[2 * T:]
        bufs = (g0, g1, g2, g3)
        c = lax.axis_index("c")
        s = lax.axis_index("s")

        def run_chunk(table, out):
            _fill_rows(g0, 0.0)
            _zero_acc_rows(acc_sh, g0, s)
            plsc.subcore_barrier()

            for span in range(AWPS // ASPAN):
                row0 = s * AWPS + span * ASPAN
                pltpu.sync_copy(src_hbm.at[pl.ds(row0, ASPAN)], src_v)
                pltpu.sync_copy(dst_hbm.at[pl.ds(row0, ASPAN)], dst_v)
                pltpu.async_copy(table.at[src_v.at[0]], g0, semg)
                pltpu.async_copy(table.at[src_v.at[1]], g1, semg)

                @pl.loop(0, ASPAN, step=4)
                def _(w):
                    for j in range(4):
                        ww = w + j
                        pltpu.make_async_copy(
                            table.at[src_v.at[0]], bufs[j], semg).wait()
                        pltpu.async_copy(
                            bufs[j], acc_sh.at[dst_v.at[ww]], sems, add=True)

                        @pl.when(ww + 2 < ASPAN)
                        def _(j=j, ww=ww):
                            @pl.when(ww >= 2)
                            def _():
                                pltpu.make_async_copy(
                                    g0, acc_sh.at[dst_v.at[0]], sems).wait()

                            pltpu.async_copy(
                                table.at[src_v.at[ww + 2]], bufs[(j + 2) % 4],
                                semg)

                for _ in range(4):
                    pltpu.make_async_copy(g0, acc_sh.at[dst_v.at[0]], sems).wait()

            plsc.subcore_barrier()
            base = s * ROWS_PER_SUB
            pltpu.sync_copy(
                acc_sh.at[pl.ds(base, ROWS_PER_SUB)],
                out.at[pl.ds(base, ROWS_PER_SUB)],
            )

        for k in range(NCORE):
            @pl.when(c == k)
            def _(k=k):
                for t in range(T // 2):
                    idx = k * (T // 2) + t
                    run_chunk(tables[idx], outrefs[idx])

    return agg


_agg2 = _make_agg(2)
_agg4 = _make_agg(4)


def _row_coeff(deg_ref, i):
    rows = i * _RB + lax.broadcasted_iota(jnp.int32, (_RB, 1), 0)
    cexp = lax.rsqrt(jnp.maximum(deg_ref[:, 0:1], 1.0))
    return jnp.where(rows < N_NODES, cexp, 0.0), rows


def _scale_split_body(x_ref, dego_ref, o0_ref, o1_ref):
    i = pl.program_id(0)
    c, _ = _row_coeff(dego_ref, i)
    xs = x_ref[...] * c
    o0_ref[...] = xs[:, :FCHUNK]
    o1_ref[...] = xs[:, FCHUNK:]


_scale_split = pl.pallas_call(
    _scale_split_body,
    grid=(NPAD // _RB,),
    in_specs=[
        pl.BlockSpec((_RB, IN_DIM), lambda i: (i, 0)),
        pl.BlockSpec((_RB, DEG_W), lambda i: (i, 0)),
    ],
    out_specs=[pl.BlockSpec((_RB, FCHUNK), lambda i: (i, 0))] * 2,
    out_shape=[jax.ShapeDtypeStruct((NPAD, FCHUNK), jnp.float32)] * 2,
)


def _l1_body(a0, a1, degi, dego, w1, b1, o0, o1, o2, o3):
    i = pl.program_id(0)
    cd, _ = _row_coeff(degi, i)
    cs, _ = _row_coeff(dego, i)
    a = jnp.concatenate([a0[...], a1[...]], axis=1) * cd
    h = jnp.dot(a, w1[...], preferred_element_type=jnp.float32) + b1[...]
    hs = jnp.maximum(h, 0.0) * cs
    for t, o in enumerate((o0, o1, o2, o3)):
        o[...] = hs[:, t * FCHUNK:(t + 1) * FCHUNK]


_l1 = pl.pallas_call(
    _l1_body,
    grid=(NPAD // _RB,),
    in_specs=[
        pl.BlockSpec((_RB, FCHUNK), lambda i: (i, 0)),
        pl.BlockSpec((_RB, FCHUNK), lambda i: (i, 0)),
        pl.BlockSpec((_RB, DEG_W), lambda i: (i, 0)),
        pl.BlockSpec((_RB, DEG_W), lambda i: (i, 0)),
        pl.BlockSpec((IN_DIM, HID), lambda i: (0, 0)),
        pl.BlockSpec((1, HID), lambda i: (0, 0)),
    ],
    out_specs=[pl.BlockSpec((_RB, FCHUNK), lambda i: (i, 0))] * 4,
    out_shape=[jax.ShapeDtypeStruct((NPAD, FCHUNK), jnp.float32)] * 4,
)


def _l2_body(a0, a1, a2, a3, degi, w2, b2, wc, bc, out_ref, acc_ref):
    i = pl.program_id(0)

    @pl.when(i == 0)
    def _():
        acc_ref[...] = jnp.zeros_like(acc_ref)

    cd, rows = _row_coeff(degi, i)
    a = jnp.concatenate([a0[...], a1[...], a2[...], a3[...]], axis=1) * cd
    h = jnp.dot(a, w2[...], preferred_element_type=jnp.float32) + b2[...]
    h = jnp.maximum(h, 0.0) * jnp.where(rows < N_NODES, 1.0, 0.0)
    acc_ref[...] += jnp.sum(h, axis=0, keepdims=True)

    @pl.when(i == pl.num_programs(0) - 1)
    def _():
        hg = acc_ref[...] * (1.0 / N_NODES)
        out_ref[...] = (
            jnp.dot(hg, wc[...], preferred_element_type=jnp.float32) + bc[...]
        )


_l2 = pl.pallas_call(
    _l2_body,
    grid=(NPAD // _RB,),
    in_specs=[
        pl.BlockSpec((_RB, FCHUNK), lambda i: (i, 0)),
        pl.BlockSpec((_RB, FCHUNK), lambda i: (i, 0)),
        pl.BlockSpec((_RB, FCHUNK), lambda i: (i, 0)),
        pl.BlockSpec((_RB, FCHUNK), lambda i: (i, 0)),
        pl.BlockSpec((_RB, DEG_W), lambda i: (i, 0)),
        pl.BlockSpec((HID, HID), lambda i: (0, 0)),
        pl.BlockSpec((1, HID), lambda i: (0, 0)),
        pl.BlockSpec((HID, N_CLS), lambda i: (0, 0)),
        pl.BlockSpec((1, N_CLS), lambda i: (0, 0)),
    ],
    out_specs=pl.BlockSpec((1, N_CLS), lambda i: (0, 0)),
    out_shape=jax.ShapeDtypeStruct((1, N_CLS), jnp.float32),
    scratch_shapes=[pltpu.VMEM((1, HID), jnp.float32)],
)


def kernel(x, edge_index, W1, b1, W2, b2, Wc, bc):
    src = edge_index[0].astype(jnp.int32)
    dst = edge_index[1].astype(jnp.int32)
    pad_n = EPAD - N_EDGES
    pad_idx = N_NODES + (jnp.arange(pad_n, dtype=jnp.int32) % (NPAD - N_NODES))
    src_flat = jnp.concatenate([src, pad_idx])
    dst_flat = jnp.concatenate([dst, pad_idx])
    src_w = src_flat.reshape(NSUB * WPS, WIN)
    dst_w = dst_flat.reshape(NSUB * WPS, WIN)
    src_a = src_flat.reshape(NSUB * AWPS, AWIN)
    dst_a = dst_flat.reshape(NSUB * AWPS, AWIN)
    x_pad = jnp.pad(x, ((0, NPAD - N_NODES), (0, 0)))

    dego, degi = _deg_kernel(src_w, dst_w)
    xs0, xs1 = _scale_split(x_pad, dego)
    a10, a11 = _agg2(src_a, dst_a, xs0, xs1)
    h0, h1, h2, h3 = _l1(a10, a11, degi, dego, W1, b1.reshape(1, HID))
    a20, a21, a22, a23 = _agg4(src_a, dst_a, h0, h1, h2, h3)
    return _l2(a20, a21, a22, a23, degi, W2, b2.reshape(1, HID),
               Wc, bc.reshape(1, N_CLS))


# speedup vs baseline: 8.2566x; 1.1427x over previous
import dataclasses
import functools

import jax
import jax.numpy as jnp
from jax import lax
from jax.experimental import pallas as pl
from jax.experimental.pallas import tpu as pltpu
from jax.experimental.pallas import tpu_sc as plsc

N_NODES = 10000
IN_DIM = 256
HID = 512
N_CLS = 64
N_EDGES = 160000

NSUB = 16
NCORE = 2
WIN = 128
WPS = 80
AWIN = 64
AWPS = 160
ASPAN = 40
EPAD = NCORE * 0 + NSUB * WPS * WIN
NPAD = 10240
ROWS_PER_SUB = NPAD // NSUB
FCHUNK = 128

_RB = 1280


def _sc_mesh():
    return plsc.VectorSubcoreMesh(core_axis_name="c", subcore_axis_name="s")


def _fill_rows(buf, value):
    rows, cols = buf.shape
    v = jnp.full((16,), value, jnp.float32)

    @pl.loop(0, rows)
    def _(i):
        for j in range(0, cols, 16):
            buf[i, pl.ds(j, 16)] = v


def _zero_acc_rows(acc_sh, zbuf, s):
    zr = zbuf.shape[0]
    base = s * ROWS_PER_SUB
    for j in range(ROWS_PER_SUB // zr):
        pltpu.sync_copy(zbuf, acc_sh.at[pl.ds(base + j * zr, zr)])
    tail = ROWS_PER_SUB % zr
    if tail:
        pltpu.sync_copy(
            zbuf.at[pl.ds(0, tail)],
            acc_sh.at[pl.ds(base + (ROWS_PER_SUB // zr) * zr, tail)],
        )


_cp_no_layout = pltpu.CompilerParams()
if "needs_layout_passes" in pltpu.CompilerParams.__dataclass_fields__:
    _cp_no_layout = dataclasses.replace(_cp_no_layout, needs_layout_passes=False)


@functools.partial(
    pl.kernel,
    compiler_params=_cp_no_layout,
    out_type=[
        jax.ShapeDtypeStruct((NSUB, NPAD), jnp.float32),
        jax.ShapeDtypeStruct((NSUB, NPAD), jnp.float32),
    ],
    mesh=_sc_mesh(),
    scratch_types=[
        pltpu.VMEM((WPS, WIN), jnp.int32),
        pltpu.VMEM((1, NPAD), jnp.float32),
    ],
)
def _deg_kernel(src_hbm, dst_hbm, dego_hbm, degi_hbm, idx_v, hist_v):
    c = lax.axis_index("c")
    s = lax.axis_index("s")

    def side(idx_hbm, out_hbm):
        pltpu.sync_copy(idx_hbm.at[pl.ds(s * WPS, WPS)], idx_v)
        vz = jnp.zeros((16,), jnp.float32)

        @pl.loop(0, NPAD // 16)
        def _(i):
            hist_v[0, pl.ds(i * 16, 16)] = vz

        ones = jnp.full((16,), 1.0, jnp.float32)

        @pl.loop(0, WPS)
        def _(w):
            for j in range(WIN // 16):
                idx = idx_v[w, pl.ds(j * 16, 16)]
                plsc.addupdate_scatter(hist_v.at[0], [idx], ones)

        pltpu.sync_copy(hist_v, out_hbm.at[pl.ds(s, 1)])

    @pl.when(c == 0)
    def _():
        side(src_hbm, dego_hbm)

    @pl.when(c == 1)
    def _():
        side(dst_hbm, degi_hbm)


def _make_agg(num_tables):
    T = num_tables

    @functools.partial(
        pl.kernel,
        out_type=[jax.ShapeDtypeStruct((NPAD, FCHUNK), jnp.float32) for _ in range(T)],
        mesh=_sc_mesh(),
        scratch_types=[
            pltpu.VMEM((ASPAN, AWIN), jnp.int32),
            pltpu.VMEM((ASPAN, AWIN), jnp.int32),
            pltpu.VMEM((AWIN, FCHUNK), jnp.float32),
            pltpu.VMEM((AWIN, FCHUNK), jnp.float32),
            pltpu.VMEM((AWIN, FCHUNK), jnp.float32),
            pltpu.VMEM((AWIN, FCHUNK), jnp.float32),
            pltpu.VMEM_SHARED((NPAD, FCHUNK), jnp.float32),
            pltpu.SemaphoreType.DMA,
            pltpu.SemaphoreType.DMA,
        ],
    )
    def agg(src_hbm, dst_hbm, *rest):
        tables = rest[:T]
        outrefs = rest[T:2 * T]
        src_v, dst_v, g0, g1, g2, g3, acc_sh, semg, sems = rest[2 * T:]
        bufs = (g0, g1, g2, g3)
        c = lax.axis_index("c")
        s = lax.axis_index("s")

        def run_chunk(table, out):
            _fill_rows(g0, 0.0)
            _zero_acc_rows(acc_sh, g0, s)
            plsc.subcore_barrier()

            for span in range(AWPS // ASPAN):
                row0 = s * AWPS + span * ASPAN
                pltpu.sync_copy(src_hbm.at[pl.ds(row0, ASPAN)], src_v)
                pltpu.sync_copy(dst_hbm.at[pl.ds(row0, ASPAN)], dst_v)
                pltpu.async_copy(table.at[src_v.at[0]], g0, semg)
                pltpu.async_copy(table.at[src_v.at[1]], g1, semg)

                @pl.loop(0, ASPAN, step=4)
                def _(w):
                    for j in range(4):
                        ww = w + j
                        pltpu.make_async_copy(
                            table.at[src_v.at[0]], bufs[j], semg).wait()
                        pltpu.async_copy(
                            bufs[j], acc_sh.at[dst_v.at[ww]], sems, add=True)

                        @pl.when(ww + 2 < ASPAN)
                        def _(j=j, ww=ww):
                            @pl.when(ww >= 2)
                            def _():
                                pltpu.make_async_copy(
                                    g0, acc_sh.at[dst_v.at[0]], sems).wait()

                            pltpu.async_copy(
                                table.at[src_v.at[ww + 2]], bufs[(j + 2) % 4],
                                semg)

                for _ in range(4):
                    pltpu.make_async_copy(g0, acc_sh.at[dst_v.at[0]], sems).wait()

            plsc.subcore_barrier()
            base = s * ROWS_PER_SUB
            pltpu.sync_copy(
                acc_sh.at[pl.ds(base, ROWS_PER_SUB)],
                out.at[pl.ds(base, ROWS_PER_SUB)],
            )

        for k in range(NCORE):
            @pl.when(c == k)
            def _(k=k):
                for t in range(T // 2):
                    idx = k * (T // 2) + t
                    run_chunk(tables[idx], outrefs[idx])

    return agg


_agg2 = _make_agg(2)
_agg4 = _make_agg(4)


def _row_coeff(degp_ref, i):
    d = jnp.sum(degp_ref[...], axis=0)
    cexp = lax.rsqrt(jnp.maximum(d, 1.0)).reshape(_RB, 1)
    rows = i * _RB + lax.broadcasted_iota(jnp.int32, (_RB, 1), 0)
    return jnp.where(rows < N_NODES, cexp, 0.0), rows


def _scale_split_body(x_ref, dego_ref, o0_ref, o1_ref):
    i = pl.program_id(0)
    c, _ = _row_coeff(dego_ref, i)
    xs = x_ref[...] * c
    o0_ref[...] = xs[:, :FCHUNK]
    o1_ref[...] = xs[:, FCHUNK:]


_scale_split = pl.pallas_call(
    _scale_split_body,
    grid=(NPAD // _RB,),
    in_specs=[
        pl.BlockSpec((_RB, IN_DIM), lambda i: (i, 0)),
        pl.BlockSpec((NSUB, _RB), lambda i: (0, i)),
    ],
    out_specs=[pl.BlockSpec((_RB, FCHUNK), lambda i: (i, 0))] * 2,
    out_shape=[jax.ShapeDtypeStruct((NPAD, FCHUNK), jnp.float32)] * 2,
)


def _l1_body(a0, a1, degi, dego, w1, b1, o0, o1, o2, o3):
    i = pl.program_id(0)
    cd, _ = _row_coeff(degi, i)
    cs, _ = _row_coeff(dego, i)
    a = jnp.concatenate([a0[...], a1[...]], axis=1) * cd
    h = jnp.dot(a, w1[...], preferred_element_type=jnp.float32) + b1[...]
    hs = jnp.maximum(h, 0.0) * cs
    for t, o in enumerate((o0, o1, o2, o3)):
        o[...] = hs[:, t * FCHUNK:(t + 1) * FCHUNK]


_l1 = pl.pallas_call(
    _l1_body,
    grid=(NPAD // _RB,),
    in_specs=[
        pl.BlockSpec((_RB, FCHUNK), lambda i: (i, 0)),
        pl.BlockSpec((_RB, FCHUNK), lambda i: (i, 0)),
        pl.BlockSpec((NSUB, _RB), lambda i: (0, i)),
        pl.BlockSpec((NSUB, _RB), lambda i: (0, i)),
        pl.BlockSpec((IN_DIM, HID), lambda i: (0, 0)),
        pl.BlockSpec((1, HID), lambda i: (0, 0)),
    ],
    out_specs=[pl.BlockSpec((_RB, FCHUNK), lambda i: (i, 0))] * 4,
    out_shape=[jax.ShapeDtypeStruct((NPAD, FCHUNK), jnp.float32)] * 4,
)


def _l2_body(a0, a1, a2, a3, degi, w2, b2, wc, bc, out_ref, acc_ref):
    i = pl.program_id(0)

    @pl.when(i == 0)
    def _():
        acc_ref[...] = jnp.zeros_like(acc_ref)

    cd, rows = _row_coeff(degi, i)
    a = jnp.concatenate([a0[...], a1[...], a2[...], a3[...]], axis=1) * cd
    h = jnp.dot(a, w2[...], preferred_element_type=jnp.float32) + b2[...]
    h = jnp.maximum(h, 0.0) * jnp.where(rows < N_NODES, 1.0, 0.0)
    acc_ref[...] += jnp.sum(h, axis=0, keepdims=True)

    @pl.when(i == pl.num_programs(0) - 1)
    def _():
        hg = acc_ref[...] * (1.0 / N_NODES)
        out_ref[...] = (
            jnp.dot(hg, wc[...], preferred_element_type=jnp.float32) + bc[...]
        )


_l2 = pl.pallas_call(
    _l2_body,
    grid=(NPAD // _RB,),
    in_specs=[
        pl.BlockSpec((_RB, FCHUNK), lambda i: (i, 0)),
        pl.BlockSpec((_RB, FCHUNK), lambda i: (i, 0)),
        pl.BlockSpec((_RB, FCHUNK), lambda i: (i, 0)),
        pl.BlockSpec((_RB, FCHUNK), lambda i: (i, 0)),
        pl.BlockSpec((NSUB, _RB), lambda i: (0, i)),
        pl.BlockSpec((HID, HID), lambda i: (0, 0)),
        pl.BlockSpec((1, HID), lambda i: (0, 0)),
        pl.BlockSpec((HID, N_CLS), lambda i: (0, 0)),
        pl.BlockSpec((1, N_CLS), lambda i: (0, 0)),
    ],
    out_specs=pl.BlockSpec((1, N_CLS), lambda i: (0, 0)),
    out_shape=jax.ShapeDtypeStruct((1, N_CLS), jnp.float32),
    scratch_shapes=[pltpu.VMEM((1, HID), jnp.float32)],
)


def kernel(x, edge_index, W1, b1, W2, b2, Wc, bc):
    src = edge_index[0].astype(jnp.int32)
    dst = edge_index[1].astype(jnp.int32)
    pad_n = EPAD - N_EDGES
    pad_idx = N_NODES + (jnp.arange(pad_n, dtype=jnp.int32) % (NPAD - N_NODES))
    src_flat = jnp.concatenate([src, pad_idx])
    dst_flat = jnp.concatenate([dst, pad_idx])
    src_w = src_flat.reshape(NSUB * WPS, WIN)
    dst_w = dst_flat.reshape(NSUB * WPS, WIN)
    src_a = src_flat.reshape(NSUB * AWPS, AWIN)
    dst_a = dst_flat.reshape(NSUB * AWPS, AWIN)
    x_pad = jnp.pad(x, ((0, NPAD - N_NODES), (0, 0)))

    dego, degi = _deg_kernel(src_w, dst_w)
    xs0, xs1 = _scale_split(x_pad, dego)
    a10, a11 = _agg2(src_a, dst_a, xs0, xs1)
    h0, h1, h2, h3 = _l1(a10, a11, degi, dego, W1, b1.reshape(1, HID))
    a20, a21, a22, a23 = _agg4(src_a, dst_a, h0, h1, h2, h3)
    return _l2(a20, a21, a22, a23, degi, W2, b2.reshape(1, HID),
               Wc, bc.reshape(1, N_CLS))
